# baseline (speedup 1.0000x reference)
import functools

import jax
import jax.numpy as jnp
from jax.experimental import pallas as pl
from jax.experimental.pallas import tpu as pltpu

_BN_EPS = 1e-5
_NB = 8


def _body(x0a_ref, x0b_ref, x1a_ref, x1b_ref, w01_ref, w02_ref, w11_ref, w12_ref,
          o0_ref, o1_ref, y_ref, acc_ref, stat_ref,
          *, p1, nb, c_half, m_pix, m_total):
    i = pl.program_id(0)

    @pl.when(i == 0)
    def _init():
        acc_ref[...] = jnp.zeros_like(acc_ref)

    @pl.when(i < p1)
    def _compute():
        for k in range(nb):
            taps = (
                (x0a_ref, w01_ref),
                (x0b_ref, w02_ref),
                (x1a_ref, w11_ref),
                (x1b_ref, w12_ref),
            )
            for g, (tap_ref, w_ref) in enumerate(taps):
                xm = tap_ref[k].reshape(m_pix, tap_ref.shape[-1])
                yg = jnp.dot(xm, w_ref[...], preferred_element_type=jnp.float32)
                y_ref[i, k, g] = yg.astype(jnp.bfloat16)
                acc_ref[0, g] += yg
                acc_ref[1, g] += yg * yg

    @pl.when(i == p1 - 1)
    def _stats():
        ssum = jnp.sum(acc_ref[0], axis=-2, keepdims=True)
        ssq = jnp.sum(acc_ref[1], axis=-2, keepdims=True)
        mean = ssum / m_total
        var = jnp.maximum(ssq / m_total - mean * mean, 0.0)
        inv = jax.lax.rsqrt(var + _BN_EPS)
        stat_ref[0] = inv
        stat_ref[1] = -mean * inv

    @pl.when(i >= p1)
    def _normalize():
        j = i - p1
        out = y_ref[j].astype(jnp.float32) * stat_ref[0] + stat_ref[1]
        o0_ref[:, :, :c_half] = out[:, 0]
        o0_ref[:, :, c_half:] = out[:, 1]
        o1_ref[:, :, :c_half] = out[:, 2]
        o1_ref[:, :, c_half:] = out[:, 3]


def kernel(pprev, prev, r0_w1, r0_w2, r1_w1, r1_w2):
    n, c0, h, w = pprev.shape
    _, c1, _, _ = prev.shape
    ho, wo = h // 2, w // 2
    m_pix = ho * wo
    c_half = r0_w1.shape[0]
    nb = _NB
    p1 = n // nb
    m_total = n * m_pix

    x0 = jnp.transpose(pprev, (0, 2, 3, 1)).reshape(n, ho, 2, wo, 2 * c0)
    x1 = jnp.transpose(prev, (0, 2, 3, 1)).reshape(n, ho, 2, wo, 2 * c1)
    w01 = r0_w1.reshape(c_half, c0).T
    w02 = r0_w2.reshape(c_half, c0).T
    w11 = r1_w1.reshape(c_half, c1).T
    w12 = r1_w2.reshape(c_half, c1).T

    body = functools.partial(_body, p1=p1, nb=nb, c_half=c_half,
                             m_pix=m_pix, m_total=m_total)
    o0, o1 = pl.pallas_call(
        body,
        out_shape=(jax.ShapeDtypeStruct((n, m_pix, 2 * c_half), jnp.float32),
                   jax.ShapeDtypeStruct((n, m_pix, 2 * c_half), jnp.float32)),
        grid=(2 * p1,),
        in_specs=[
            pl.BlockSpec((nb, ho, 1, wo, c0), lambda i: (jnp.minimum(i, p1 - 1), 0, 0, 0, 0)),
            pl.BlockSpec((nb, ho, 1, wo, c0), lambda i: (jnp.minimum(i, p1 - 1), 0, 1, 0, 1)),
            pl.BlockSpec((nb, ho, 1, wo, c1), lambda i: (jnp.minimum(i, p1 - 1), 0, 0, 0, 0)),
            pl.BlockSpec((nb, ho, 1, wo, c1), lambda i: (jnp.minimum(i, p1 - 1), 0, 1, 0, 1)),
            pl.BlockSpec((c0, c_half), lambda i: (0, 0)),
            pl.BlockSpec((c0, c_half), lambda i: (0, 0)),
            pl.BlockSpec((c1, c_half), lambda i: (0, 0)),
            pl.BlockSpec((c1, c_half), lambda i: (0, 0)),
        ],
        out_specs=(
            pl.BlockSpec((nb, m_pix, 2 * c_half), lambda i: (jnp.maximum(i - p1, 0), 0, 0)),
            pl.BlockSpec((nb, m_pix, 2 * c_half), lambda i: (jnp.maximum(i - p1, 0), 0, 0)),
        ),
        scratch_shapes=[
            pltpu.VMEM((p1, nb, 4, m_pix, c_half), jnp.bfloat16),
            pltpu.VMEM((2, 4, m_pix, c_half), jnp.float32),
            pltpu.VMEM((2, 4, 1, c_half), jnp.float32),
        ],
        compiler_params=pltpu.CompilerParams(
            dimension_semantics=("arbitrary",),
            vmem_limit_bytes=52 * 1024 * 1024),
        name="fused_reduction_layer",
    )(x0, x0, x1, x1, w01, w02, w11, w12)

    o0 = jnp.transpose(o0.reshape(n, ho, wo, 2 * c_half), (0, 3, 1, 2))
    o1 = jnp.transpose(o1.reshape(n, ho, wo, 2 * c_half), (0, 3, 1, 2))
    return o0, o1


# speedup vs baseline: 2.7535x; 2.7535x over previous
import functools

import jax
import jax.numpy as jnp
from jax.experimental import pallas as pl
from jax.experimental.pallas import tpu as pltpu

_BN_EPS = 1e-5
_NB = 8


def _body(x0_ref, x1_ref, w01_ref, w02_ref, w11_ref, w12_ref,
          o0_ref, o1_ref, y_ref, acc_ref, stat_ref,
          *, p1, nb, c_half, m_pix, m_total):
    i = pl.program_id(0)

    @pl.when(i == 0)
    def _init():
        acc_ref[...] = jnp.zeros_like(acc_ref)

    @pl.when(i < p1)
    def _compute():
        for k in range(nb):
            taps = (
                (x0_ref[k, 0::2, 0::2, :], w01_ref),
                (x0_ref[k, 1::2, 1::2, :], w02_ref),
                (x1_ref[k, 0::2, 0::2, :], w11_ref),
                (x1_ref[k, 1::2, 1::2, :], w12_ref),
            )
            for g, (tap, w_ref) in enumerate(taps):
                xm = tap.reshape(m_pix, tap.shape[-1])
                yg = jnp.dot(xm, w_ref[...], preferred_element_type=jnp.float32)
                y_ref[i, k, g] = yg.astype(jnp.bfloat16)
                acc_ref[0, g] += yg
                acc_ref[1, g] += yg * yg

    @pl.when(i == p1 - 1)
    def _stats():
        ssum = jnp.sum(acc_ref[0], axis=-2, keepdims=True)
        ssq = jnp.sum(acc_ref[1], axis=-2, keepdims=True)
        mean = ssum / m_total
        var = jnp.maximum(ssq / m_total - mean * mean, 0.0)
        inv = jax.lax.rsqrt(var + _BN_EPS)
        stat_ref[0] = inv
        stat_ref[1] = -mean * inv

    @pl.when(i >= p1)
    def _normalize():
        j = i - p1
        for half in range(2):
            out = (y_ref[2 * j + half].astype(jnp.float32) * stat_ref[0]
                   + stat_ref[1])
            rows = pl.ds(half * nb, nb)
            o0_ref[rows, :, :c_half] = out[:, 0]
            o0_ref[rows, :, c_half:] = out[:, 1]
            o1_ref[rows, :, :c_half] = out[:, 2]
            o1_ref[rows, :, c_half:] = out[:, 3]


def kernel(pprev, prev, r0_w1, r0_w2, r1_w1, r1_w2):
    n, c0, h, w = pprev.shape
    _, c1, _, _ = prev.shape
    ho, wo = h // 2, w // 2
    m_pix = ho * wo
    c_half = r0_w1.shape[0]
    nb = _NB
    p1 = n // nb
    m_total = n * m_pix

    x0 = jnp.transpose(pprev, (0, 2, 3, 1))
    x1 = jnp.transpose(prev, (0, 2, 3, 1))
    w01 = r0_w1.reshape(c_half, c0).T
    w02 = r0_w2.reshape(c_half, c0).T
    w11 = r1_w1.reshape(c_half, c1).T
    w12 = r1_w2.reshape(c_half, c1).T

    body = functools.partial(_body, p1=p1, nb=nb, c_half=c_half,
                             m_pix=m_pix, m_total=m_total)
    o0, o1 = pl.pallas_call(
        body,
        out_shape=(jax.ShapeDtypeStruct((n, m_pix, 2 * c_half), jnp.float32),
                   jax.ShapeDtypeStruct((n, m_pix, 2 * c_half), jnp.float32)),
        grid=(p1 + p1 // 2,),
        in_specs=[
            pl.BlockSpec((nb, h, w, c0), lambda i: (jnp.minimum(i, p1 - 1), 0, 0, 0)),
            pl.BlockSpec((nb, h, w, c1), lambda i: (jnp.minimum(i, p1 - 1), 0, 0, 0)),
            pl.BlockSpec((c0, c_half), lambda i: (0, 0)),
            pl.BlockSpec((c0, c_half), lambda i: (0, 0)),
            pl.BlockSpec((c1, c_half), lambda i: (0, 0)),
            pl.BlockSpec((c1, c_half), lambda i: (0, 0)),
        ],
        out_specs=(
            pl.BlockSpec((2 * nb, m_pix, 2 * c_half), lambda i: (jnp.maximum(i - p1, 0), 0, 0)),
            pl.BlockSpec((2 * nb, m_pix, 2 * c_half), lambda i: (jnp.maximum(i - p1, 0), 0, 0)),
        ),
        scratch_shapes=[
            pltpu.VMEM((p1, nb, 4, m_pix, c_half), jnp.bfloat16),
            pltpu.VMEM((2, 4, m_pix, c_half), jnp.float32),
            pltpu.VMEM((2, 4, 1, c_half), jnp.float32),
        ],
        compiler_params=pltpu.CompilerParams(
            dimension_semantics=("arbitrary",),
            vmem_limit_bytes=52 * 1024 * 1024),
        name="fused_reduction_layer",
    )(x0, x1, w01, w02, w11, w12)

    o0 = jnp.transpose(o0.reshape(n, ho, wo, 2 * c_half), (0, 3, 1, 2))
    o1 = jnp.transpose(o1.reshape(n, ho, wo, 2 * c_half), (0, 3, 1, 2))
    return o0, o1


# speedup vs baseline: 2.8089x; 1.0201x over previous
import functools

import jax
import jax.numpy as jnp
from jax.experimental import pallas as pl
from jax.experimental.pallas import tpu as pltpu

_BN_EPS = 1e-5
_NB = 8


def _body(x0_ref, x1_ref, w01_ref, w02_ref, w11_ref, w12_ref,
          o0_ref, o1_ref, y_ref, acc_ref, stat_ref,
          *, p1, nb, c_half, m_pix, m_total):
    i = pl.program_id(0)

    @pl.when(i == 0)
    def _init():
        acc_ref[...] = jnp.zeros_like(acc_ref)

    @pl.when(i < p1)
    def _compute():
        for k in range(nb):
            taps = (
                (x0_ref[k, 0::2, 0::2, :], w01_ref),
                (x0_ref[k, 1::2, 1::2, :], w02_ref),
                (x1_ref[k, 0::2, 0::2, :], w11_ref),
                (x1_ref[k, 1::2, 1::2, :], w12_ref),
            )
            for g, (tap, w_ref) in enumerate(taps):
                xm = tap.reshape(m_pix, tap.shape[-1])
                yg = jnp.dot(xm, w_ref[...], preferred_element_type=jnp.float32)
                y_ref[i, k, g] = yg.astype(jnp.bfloat16)
                acc_ref[0, g] += yg
                acc_ref[1, g] += yg * yg

    @pl.when(i == p1 - 1)
    def _stats():
        ssum = jnp.sum(acc_ref[0], axis=-2, keepdims=True)
        ssq = jnp.sum(acc_ref[1], axis=-2, keepdims=True)
        mean = ssum / m_total
        var = jnp.maximum(ssq / m_total - mean * mean, 0.0)
        inv = jax.lax.rsqrt(var + _BN_EPS)
        stat_ref[0] = inv
        stat_ref[1] = -mean * inv

    @pl.when(i >= p1)
    def _normalize():
        j = i - p1
        out = y_ref[j].astype(jnp.float32) * stat_ref[0] + stat_ref[1]
        o0_ref[:, :, :c_half] = out[:, 0]
        o0_ref[:, :, c_half:] = out[:, 1]
        o1_ref[:, :, :c_half] = out[:, 2]
        o1_ref[:, :, c_half:] = out[:, 3]


def kernel(pprev, prev, r0_w1, r0_w2, r1_w1, r1_w2):
    n, c0, h, w = pprev.shape
    _, c1, _, _ = prev.shape
    ho, wo = h // 2, w // 2
    m_pix = ho * wo
    c_half = r0_w1.shape[0]
    nb = _NB
    p1 = n // nb
    m_total = n * m_pix

    x0 = jnp.transpose(pprev, (0, 2, 3, 1))
    x1 = jnp.transpose(prev, (0, 2, 3, 1))
    w01 = r0_w1.reshape(c_half, c0).T
    w02 = r0_w2.reshape(c_half, c0).T
    w11 = r1_w1.reshape(c_half, c1).T
    w12 = r1_w2.reshape(c_half, c1).T

    body = functools.partial(_body, p1=p1, nb=nb, c_half=c_half,
                             m_pix=m_pix, m_total=m_total)
    o0, o1 = pl.pallas_call(
        body,
        out_shape=(jax.ShapeDtypeStruct((n, m_pix, 2 * c_half), jnp.float32),
                   jax.ShapeDtypeStruct((n, m_pix, 2 * c_half), jnp.float32)),
        grid=(2 * p1,),
        in_specs=[
            pl.BlockSpec((nb, h, w, c0), lambda i: (jnp.minimum(i, p1 - 1), 0, 0, 0)),
            pl.BlockSpec((nb, h, w, c1), lambda i: (jnp.minimum(i, p1 - 1), 0, 0, 0)),
            pl.BlockSpec((c0, c_half), lambda i: (0, 0)),
            pl.BlockSpec((c0, c_half), lambda i: (0, 0)),
            pl.BlockSpec((c1, c_half), lambda i: (0, 0)),
            pl.BlockSpec((c1, c_half), lambda i: (0, 0)),
        ],
        out_specs=(
            pl.BlockSpec((nb, m_pix, 2 * c_half), lambda i: (jnp.maximum(i - p1, 0), 0, 0)),
            pl.BlockSpec((nb, m_pix, 2 * c_half), lambda i: (jnp.maximum(i - p1, 0), 0, 0)),
        ),
        scratch_shapes=[
            pltpu.VMEM((p1, nb, 4, m_pix, c_half), jnp.bfloat16),
            pltpu.VMEM((2, 4, m_pix, c_half), jnp.float32),
            pltpu.VMEM((2, 4, 1, c_half), jnp.float32),
        ],
        compiler_params=pltpu.CompilerParams(
            dimension_semantics=("arbitrary",),
            vmem_limit_bytes=52 * 1024 * 1024),
        name="fused_reduction_layer",
    )(x0, x1, w01, w02, w11, w12)

    o0 = jnp.transpose(o0.reshape(n, ho, wo, 2 * c_half), (0, 3, 1, 2))
    o1 = jnp.transpose(o1.reshape(n, ho, wo, 2 * c_half), (0, 3, 1, 2))
    return o0, o1
